# Initial kernel scaffold; baseline (speedup 1.0000x reference)
import jax
import jax.numpy as jnp
from jax.experimental import pallas as pl


def kernel(x_n1, x_n2, edge_index_n1_n2, edge_index_n2_n1, edge_index, lin_n1_W, lin_n1_b, lin_n2_W, lin_n2_b, convs, lin_bias_n1_W, lin_bias_n1_b, lin_bias_n2_W, lin_bias_n2_b, convs_bias, x_bias_n1, x_bias_n2, att_bias):
    raise NotImplementedError("write your pallas kernel here")


# speedup vs baseline: 1.0631x; 1.0631x over previous
import functools
import math

import jax
import jax.numpy as jnp
from jax import lax
from jax.experimental import pallas as pl
from jax.experimental.pallas import tpu as pltpu

N1 = 4096
N2 = 4096
D_FEAT = 128
C = 128
H = 4
DH = C // H
L = 2
HB = 64
E = 65536
EP = 16384
NODE_TYPES = ("n1", "n2")
EDGE_TYPES = (("n1", "n2"), ("n2", "n1"))
NUM_N = {"n1": N1, "n2": N2}


def _matmul_body(x_ref, w_ref, b_ref, o_ref, *, act):
    y = jnp.dot(x_ref[...], w_ref[...], preferred_element_type=jnp.float32)
    y = y + b_ref[...]
    if act == "relu":
        y = jnp.maximum(y, 0.0)
    o_ref[...] = y


def _linear(x, w, b, act=None, bm=512):
    n, d = x.shape
    dout = w.shape[1]
    return pl.pallas_call(
        functools.partial(_matmul_body, act=act),
        grid=(n // bm,),
        in_specs=[
            pl.BlockSpec((bm, d), lambda i: (i, 0)),
            pl.BlockSpec((d, dout), lambda i: (0, 0)),
            pl.BlockSpec((dout,), lambda i: (0,)),
        ],
        out_specs=pl.BlockSpec((bm, dout), lambda i: (i, 0)),
        out_shape=jax.ShapeDtypeStruct((n, dout), jnp.float32),
    )(x, w, b)


def _final_mm_body(a_ref, b_ref, o_ref):
    o_ref[...] = lax.dot_general(
        a_ref[...], b_ref[...], (((1,), (1,)), ((), ())),
        preferred_element_type=jnp.float32)


def _final_matmul(em, ed, bm=1024, bn=1024):
    n, k = em.shape
    m2 = ed.shape[0]
    return pl.pallas_call(
        _final_mm_body,
        grid=(n // bm, m2 // bn),
        in_specs=[
            pl.BlockSpec((bm, k), lambda i, j: (i, 0)),
            pl.BlockSpec((bn, k), lambda i, j: (j, 0)),
        ],
        out_specs=pl.BlockSpec((bm, bn), lambda i, j: (i, j)),
        out_shape=jax.ShapeDtypeStruct((n, m2), jnp.float32),
    )(em, ed)


def _fused_proj_weights(p, nt, et):
    Wq, bq = p[nt + "_q_W"], p[nt + "_q_b"]
    Wk, bk = p[nt + "_k_W"], p[nt + "_k_b"]
    Wv, bv = p[nt + "_v_W"], p[nt + "_v_b"]
    a_rel, m_rel, p_rel = p[et + "_a_rel"], p[et + "_m_rel"], p[et + "_p_rel"]
    scale = p_rel / math.sqrt(float(DH))
    Wkr = jnp.concatenate(
        [Wk[:, h*DH:(h+1)*DH] @ (a_rel[h] * scale[h]) for h in range(H)], axis=1)
    bkr = jnp.concatenate(
        [bk[h*DH:(h+1)*DH] @ (a_rel[h] * scale[h]) for h in range(H)])
    Wvr = jnp.concatenate(
        [Wv[:, h*DH:(h+1)*DH] @ m_rel[h] for h in range(H)], axis=1)
    bvr = jnp.concatenate([bv[h*DH:(h+1)*DH] @ m_rel[h] for h in range(H)])
    return jnp.concatenate([Wq, Wkr, Wvr], axis=1), jnp.concatenate([bq, bkr, bvr])


def _conv(x_dict, edge_dict, p):
    W1, b1 = _fused_proj_weights(p, "n1", "n1__n2")
    W2, b2 = _fused_proj_weights(p, "n2", "n2__n1")
    P = {"n1": _linear(x_dict["n1"], W1, b1),
         "n2": _linear(x_dict["n2"], W2, b2)}
    agg = {}
    for (src, dst) in EDGE_TYPES:
        si, di = edge_dict[(src, dst)][0], edge_dict[(src, dst)][1]
        q = P[dst][:, :C].reshape(-1, H, DH)
        kr = P[src][:, C:2*C].reshape(-1, H, DH)
        vr = P[src][:, 2*C:].reshape(-1, H, DH)
        alpha = (q[di] * kr[si]).sum(-1)
        M = alpha.max(axis=0)
        ex = jnp.exp(alpha - M[None, :])
        msg = vr[si] * ex[:, :, None]
        n_dst = NUM_N[dst]
        U = jax.ops.segment_sum(msg, di, num_segments=n_dst)
        den = jax.ops.segment_sum(ex, di, num_segments=n_dst)
        agg[dst] = U / (den[:, :, None] + 1e-16)
    out = {}
    for nt in NODE_TYPES:
        o = jax.nn.gelu(agg[nt].reshape(NUM_N[nt], C))
        o = _linear(o, p[nt + "_a_W"], p[nt + "_a_b"])
        beta = jax.nn.sigmoid(p[nt + "_skip"])
        out[nt] = beta * o + (1.0 - beta) * x_dict[nt]
    return out


def kernel(x_n1, x_n2, edge_index_n1_n2, edge_index_n2_n1, edge_index,
           lin_n1_W, lin_n1_b, lin_n2_W, lin_n2_b, convs,
           lin_bias_n1_W, lin_bias_n1_b, lin_bias_n2_W, lin_bias_n2_b,
           convs_bias, x_bias_n1, x_bias_n2, att_bias):
    edge_dict = {("n1", "n2"): edge_index_n1_n2, ("n2", "n1"): edge_index_n2_n1}
    x = {"n1": _linear(x_n1, lin_n1_W, lin_n1_b, act="relu"),
         "n2": _linear(x_n2, lin_n2_W, lin_n2_b, act="relu")}
    allm = []
    for li in range(L):
        x = _conv(x, edge_dict, convs[li])
        allm.append(x)
    xb = {"n1": _linear(x_bias_n1, lin_bias_n1_W, lin_bias_n1_b, act="relu"),
          "n2": _linear(x_bias_n2, lin_bias_n2_W, lin_bias_n2_b, act="relu")}
    allb = []
    for li in range(L):
        xb = _conv(xb, edge_dict, convs_bias[li])
        allb.append(xb)
    ab = att_bias
    fin = {}
    for nt in NODE_TYPES:
        cm = jnp.concatenate([t[nt] for t in allm], axis=1)
        cb = jnp.concatenate([t[nt] for t in allb], axis=1)
        fin[nt] = ab[0] * cm + ab[1] * cb
    y_all = _final_matmul(fin["n1"], fin["n2"])
    y = y_all[edge_index[0], edge_index[1]][:, None]
    return y, y_all


# speedup vs baseline: 24.0531x; 22.6256x over previous
import functools
import math

import jax
import jax.numpy as jnp
from jax import lax
from jax.experimental import pallas as pl
from jax.experimental.pallas import tpu as pltpu
from jax.experimental.pallas import tpu_sc as plsc

N1 = 4096
N2 = 4096
D_FEAT = 128
C = 128
H = 4
DH = C // H
L = 2
HB = 64
E = 65536
EP = 16384
NODE_TYPES = ("n1", "n2")
EDGE_TYPES = (("n1", "n2"), ("n2", "n1"))
NUM_N = {"n1": N1, "n2": N2}


def _matmul_body(x_ref, w_ref, b_ref, o_ref, *, act):
    y = jnp.dot(x_ref[...], w_ref[...], preferred_element_type=jnp.float32)
    y = y + b_ref[...]
    if act == "relu":
        y = jnp.maximum(y, 0.0)
    o_ref[...] = y


def _linear(x, w, b, act=None, bm=512):
    n, d = x.shape
    dout = w.shape[1]
    return pl.pallas_call(
        functools.partial(_matmul_body, act=act),
        grid=(n // bm,),
        in_specs=[
            pl.BlockSpec((bm, d), lambda i: (i, 0)),
            pl.BlockSpec((d, dout), lambda i: (0, 0)),
            pl.BlockSpec((dout,), lambda i: (0,)),
        ],
        out_specs=pl.BlockSpec((bm, dout), lambda i: (i, 0)),
        out_shape=jax.ShapeDtypeStruct((n, dout), jnp.float32),
    )(x, w, b)


def _final_mm_body(a_ref, b_ref, o_ref):
    o_ref[...] = lax.dot_general(
        a_ref[...], b_ref[...], (((1,), (1,)), ((), ())),
        preferred_element_type=jnp.float32)


def _final_matmul(em, ed, bm=1024, bn=1024):
    n, k = em.shape
    m2 = ed.shape[0]
    return pl.pallas_call(
        _final_mm_body,
        grid=(n // bm, m2 // bn),
        in_specs=[
            pl.BlockSpec((bm, k), lambda i, j: (i, 0)),
            pl.BlockSpec((bn, k), lambda i, j: (j, 0)),
        ],
        out_specs=pl.BlockSpec((bm, bn), lambda i, j: (i, j)),
        out_shape=jax.ShapeDtypeStruct((n, m2), jnp.float32),
    )(em, ed)


_SC_MESH = dict(core_axis_name="c", subcore_axis_name="s")
_NW = 32


def _sc_scalar_store(ref, i, s):
    lanes = lax.iota(jnp.int32, 16)
    plsc.store_scatter(ref, [jnp.full((16,), i, jnp.int32)],
                       jnp.full((16,), s, jnp.float32), mask=lanes == 0)


def _pair_dot_body(fin_hbm, eip_hbm, y_hbm, aidx, bidx, arows, brows, ybuf,
                   sem1, sem2, *, ch, nch):
    wid = lax.axis_index("s") * 2 + lax.axis_index("c")
    base = wid * (ch * nch)

    def chunk(ci, _):
        cb = base + ci * ch
        pltpu.sync_copy(eip_hbm.at[0, pl.ds(cb, ch)], aidx)
        pltpu.sync_copy(eip_hbm.at[1, pl.ds(cb, ch)], bidx)

        def adj(g, _):
            bidx[pl.ds(g * 16, 16)] = bidx[pl.ds(g * 16, 16)] + 4096
            return 0
        lax.fori_loop(0, ch // 16, adj, 0, unroll=True)
        cp1 = pltpu.async_copy(fin_hbm.at[aidx], arows, sem1)
        cp2 = pltpu.async_copy(fin_hbm.at[bidx], brows, sem2)
        cp1.wait()
        cp2.wait()

        def dot1(e, _):
            acc = arows[e, pl.ds(0, 16)] * brows[e, pl.ds(0, 16)]
            for j in range(1, 16):
                acc = acc + arows[e, pl.ds(j * 16, 16)] * brows[e, pl.ds(j * 16, 16)]
            _sc_scalar_store(ybuf, e, jnp.sum(acc))
            return 0
        lax.fori_loop(0, ch, dot1, 0)
        pltpu.sync_copy(ybuf, y_hbm.at[pl.ds(cb, ch)])
        return 0
    lax.fori_loop(0, nch, chunk, 0)


def _pair_dot(fin_flat, eip):
    ep = eip.shape[1]
    ch = 128
    nch = ep // (_NW * ch)
    import functools as _ft
    f = _ft.partial(
        pl.kernel,
        out_type=jax.ShapeDtypeStruct((ep,), jnp.float32),
        mesh=plsc.VectorSubcoreMesh(**_SC_MESH),
        compiler_params=pltpu.CompilerParams(needs_layout_passes=False),
        scratch_types=[
            pltpu.VMEM((ch,), jnp.int32),
            pltpu.VMEM((ch,), jnp.int32),
            pltpu.VMEM((ch, 256), jnp.float32),
            pltpu.VMEM((ch, 256), jnp.float32),
            pltpu.VMEM((ch,), jnp.float32),
            pltpu.SemaphoreType.DMA,
            pltpu.SemaphoreType.DMA,
        ],
    )(_ft.partial(_pair_dot_body, ch=ch, nch=nch))
    return f(fin_flat, eip)


def _gather3_body(T_hbm, EI_hbm, G_hbm, sidx, didx, gidx, rows, sem, *, epw, ch):
    c = lax.axis_index("c")
    s = lax.axis_index("s")
    src_base = c * 12288
    dst_base = (1 - c) * 12288
    ebase = s * epw
    nch = epw // ch
    for role in range(3):
        def chunk(ci, _):
            cb = ebase + ci * ch
            pltpu.sync_copy(EI_hbm.at[c, 0, pl.ds(cb, ch)], sidx)
            pltpu.sync_copy(EI_hbm.at[c, 1, pl.ds(cb, ch)], didx)

            def bld(g, _):
                sl = pl.ds(g * 16, 16)
                if role == 0:
                    gidx[sl] = didx[sl] * 3 + dst_base
                else:
                    gidx[sl] = sidx[sl] * 3 + (src_base + role)
                return 0
            lax.fori_loop(0, ch // 16, bld, 0, unroll=True)
            pltpu.async_copy(T_hbm.at[gidx], rows, sem).wait()
            pltpu.sync_copy(rows, G_hbm.at[role, c, pl.ds(cb, ch)])
            return 0
        lax.fori_loop(0, nch, chunk, 0)


def _gather3(T, EI):
    epw = E // 16
    ch = 128
    f = functools.partial(
        pl.kernel,
        out_type=jax.ShapeDtypeStruct((3, 2, E, C), jnp.float32),
        mesh=plsc.VectorSubcoreMesh(**_SC_MESH),
        compiler_params=pltpu.CompilerParams(needs_layout_passes=False),
        scratch_types=[
            pltpu.VMEM((ch,), jnp.int32),
            pltpu.VMEM((ch,), jnp.int32),
            pltpu.VMEM((ch,), jnp.int32),
            pltpu.VMEM((ch, C), jnp.float32),
            pltpu.SemaphoreType.DMA,
        ],
    )(functools.partial(_gather3_body, epw=epw, ch=ch))
    return f(T, EI)


def _scatter_md(MSG, DEN, EI):
    epw = E // 16
    ch = 128
    w = 128

    def body(MSG_hbm, DEN_hbm, EI_hbm, U_hbm, D_hbm, didx, mchunk, dchunk,
             U_sh, D_sh, sem):
        c = lax.axis_index("c")
        s = lax.axis_index("s")
        ebase = s * epw
        nch = epw // ch
        zero16 = jnp.zeros((16,), jnp.float32)

        def zrow(r, _):
            for j in range(w // 16):
                mchunk[r, pl.ds(j * 16, 16)] = zero16
            return 0
        lax.fori_loop(0, ch, zrow, 0)
        rows_per_tile = 4096 // 16
        for k in range(rows_per_tile // ch):
            pltpu.sync_copy(mchunk, U_sh.at[pl.ds(s * rows_per_tile + k * ch, ch)])
            pltpu.sync_copy(mchunk, D_sh.at[pl.ds(s * rows_per_tile + k * ch, ch)])
        plsc.subcore_barrier()

        def chunk(ci, _):
            cb = ebase + ci * ch
            pltpu.sync_copy(MSG_hbm.at[c, pl.ds(cb, ch)], mchunk)
            pltpu.sync_copy(DEN_hbm.at[c, pl.ds(cb, ch)], dchunk)
            pltpu.sync_copy(EI_hbm.at[c, 1, pl.ds(cb, ch)], didx)
            pltpu.sync_copy(mchunk, U_sh.at[didx], add=True)
            pltpu.sync_copy(dchunk, D_sh.at[didx], add=True)
            return 0
        lax.fori_loop(0, nch, chunk, 0)

        plsc.subcore_barrier()
        for k in range(rows_per_tile // ch):
            off = s * rows_per_tile + k * ch
            pltpu.sync_copy(U_sh.at[pl.ds(off, ch)], U_hbm.at[c, pl.ds(off, ch)])
            pltpu.sync_copy(D_sh.at[pl.ds(off, ch)], D_hbm.at[c, pl.ds(off, ch)])

    f = pl.kernel(
        body,
        out_type=(jax.ShapeDtypeStruct((2, 4096, w), jnp.float32),
                  jax.ShapeDtypeStruct((2, 4096, w), jnp.float32)),
        mesh=plsc.VectorSubcoreMesh(**_SC_MESH),
        compiler_params=pltpu.CompilerParams(needs_layout_passes=False),
        scratch_types=[
            pltpu.VMEM((ch,), jnp.int32),
            pltpu.VMEM((ch, w), jnp.float32),
            pltpu.VMEM((ch, w), jnp.float32),
            pltpu.VMEM_SHARED((4096, w), jnp.float32),
            pltpu.VMEM_SHARED((4096, w), jnp.float32),
            pltpu.SemaphoreType.DMA,
        ],
    )
    return f(MSG, DEN, EI)


def _fused_proj_weights(p, nt, et):
    Wq, bq = p[nt + "_q_W"], p[nt + "_q_b"]
    Wk, bk = p[nt + "_k_W"], p[nt + "_k_b"]
    Wv, bv = p[nt + "_v_W"], p[nt + "_v_b"]
    a_rel, m_rel, p_rel = p[et + "_a_rel"], p[et + "_m_rel"], p[et + "_p_rel"]
    scale = p_rel / math.sqrt(float(DH))
    Wkr = jnp.concatenate(
        [Wk[:, h*DH:(h+1)*DH] @ (a_rel[h] * scale[h]) for h in range(H)], axis=1)
    bkr = jnp.concatenate(
        [bk[h*DH:(h+1)*DH] @ (a_rel[h] * scale[h]) for h in range(H)])
    Wvr = jnp.concatenate(
        [Wv[:, h*DH:(h+1)*DH] @ m_rel[h] for h in range(H)], axis=1)
    bvr = jnp.concatenate([bv[h*DH:(h+1)*DH] @ m_rel[h] for h in range(H)])
    return jnp.concatenate([Wq, Wkr, Wvr], axis=1), jnp.concatenate([bq, bkr, bvr])


def _conv(x_dict, EI, p):
    W1, b1 = _fused_proj_weights(p, "n1", "n1__n2")
    W2, b2 = _fused_proj_weights(p, "n2", "n2__n1")
    P1 = _linear(x_dict["n1"], W1, b1)
    P2 = _linear(x_dict["n2"], W2, b2)
    T = jnp.concatenate([P1, P2], axis=0).reshape(6 * 4096, C)
    G = _gather3(T, EI)
    QE, KE, VE = G[0], G[1], G[2]
    alpha = (QE * KE).reshape(2, E, H, DH).sum(-1)
    M = alpha.max(axis=1)
    ex = jnp.exp(alpha - M[:, None, :])
    msg = (VE.reshape(2, E, H, DH) * ex[..., None]).reshape(2, E, C)
    den = jnp.concatenate([ex, jnp.zeros((2, E, C - H), jnp.float32)], axis=-1)
    U, D = _scatter_md(msg, den, EI)
    out = {}
    for nt, et in (("n1", 1), ("n2", 0)):
        agg = U[et] / (jnp.repeat(D[et, :, :H], DH, axis=1) + 1e-16)
        o = jax.nn.gelu(agg)
        o = _linear(o, p[nt + "_a_W"], p[nt + "_a_b"])
        beta = jax.nn.sigmoid(p[nt + "_skip"])
        out[nt] = beta * o + (1.0 - beta) * x_dict[nt]
    return out


def kernel(x_n1, x_n2, edge_index_n1_n2, edge_index_n2_n1, edge_index,
           lin_n1_W, lin_n1_b, lin_n2_W, lin_n2_b, convs,
           lin_bias_n1_W, lin_bias_n1_b, lin_bias_n2_W, lin_bias_n2_b,
           convs_bias, x_bias_n1, x_bias_n2, att_bias):
    EI = jnp.stack([edge_index_n1_n2, edge_index_n2_n1])
    x = {"n1": _linear(x_n1, lin_n1_W, lin_n1_b, act="relu"),
         "n2": _linear(x_n2, lin_n2_W, lin_n2_b, act="relu")}
    allm = []
    for li in range(L):
        x = _conv(x, EI, convs[li])
        allm.append(x)
    xb = {"n1": _linear(x_bias_n1, lin_bias_n1_W, lin_bias_n1_b, act="relu"),
          "n2": _linear(x_bias_n2, lin_bias_n2_W, lin_bias_n2_b, act="relu")}
    allb = []
    for li in range(L):
        xb = _conv(xb, EI, convs_bias[li])
        allb.append(xb)
    ab = att_bias
    fin = {}
    for nt in NODE_TYPES:
        cm = jnp.concatenate([t[nt] for t in allm], axis=1)
        cb = jnp.concatenate([t[nt] for t in allb], axis=1)
        fin[nt] = ab[0] * cm + ab[1] * cb
    y_all = _final_matmul(fin["n1"], fin["n2"])
    fin_flat = jnp.concatenate([fin["n1"], fin["n2"]], axis=0)
    y = _pair_dot(fin_flat, edge_index).reshape(EP, 1)
    return y, y_all


# speedup vs baseline: 27.8894x; 1.1595x over previous
import functools
import math

import jax
import jax.numpy as jnp
from jax import lax
from jax.experimental import pallas as pl
from jax.experimental.pallas import tpu as pltpu
from jax.experimental.pallas import tpu_sc as plsc

N1 = 4096
N2 = 4096
D_FEAT = 128
C = 128
H = 4
DH = C // H
L = 2
HB = 64
E = 65536
EP = 16384
NODE_TYPES = ("n1", "n2")
EDGE_TYPES = (("n1", "n2"), ("n2", "n1"))
NUM_N = {"n1": N1, "n2": N2}


def _matmul_body(x_ref, w_ref, b_ref, o_ref, *, act):
    y = jnp.dot(x_ref[...], w_ref[...], preferred_element_type=jnp.float32)
    y = y + b_ref[...]
    if act == "relu":
        y = jnp.maximum(y, 0.0)
    o_ref[...] = y


def _linear(x, w, b, act=None, bm=512):
    n, d = x.shape
    dout = w.shape[1]
    return pl.pallas_call(
        functools.partial(_matmul_body, act=act),
        grid=(n // bm,),
        in_specs=[
            pl.BlockSpec((bm, d), lambda i: (i, 0)),
            pl.BlockSpec((d, dout), lambda i: (0, 0)),
            pl.BlockSpec((dout,), lambda i: (0,)),
        ],
        out_specs=pl.BlockSpec((bm, dout), lambda i: (i, 0)),
        out_shape=jax.ShapeDtypeStruct((n, dout), jnp.float32),
    )(x, w, b)


def _final_mm_body(a_ref, b_ref, o_ref):
    o_ref[...] = lax.dot_general(
        a_ref[...], b_ref[...], (((1,), (1,)), ((), ())),
        preferred_element_type=jnp.float32)


def _final_matmul(em, ed, bm=1024, bn=1024):
    n, k = em.shape
    m2 = ed.shape[0]
    return pl.pallas_call(
        _final_mm_body,
        grid=(n // bm, m2 // bn),
        in_specs=[
            pl.BlockSpec((bm, k), lambda i, j: (i, 0)),
            pl.BlockSpec((bn, k), lambda i, j: (j, 0)),
        ],
        out_specs=pl.BlockSpec((bm, bn), lambda i, j: (i, j)),
        out_shape=jax.ShapeDtypeStruct((n, m2), jnp.float32),
    )(em, ed)


_SC_MESH = dict(core_axis_name="c", subcore_axis_name="s")
_NW = 32


def _sc_scalar_store(ref, i, s):
    lanes = lax.iota(jnp.int32, 16)
    plsc.store_scatter(ref, [jnp.full((16,), i, jnp.int32)],
                       jnp.full((16,), s, jnp.float32), mask=lanes == 0)


def _pair_dot_body(fin_hbm, eip_hbm, y_hbm, aidx, bidx, arows, brows, ybuf,
                   sem1, sem2, *, ch, nch):
    wid = lax.axis_index("s") * 2 + lax.axis_index("c")
    base = wid * (ch * nch)

    def chunk(ci, _):
        cb = base + ci * ch
        pltpu.sync_copy(eip_hbm.at[0, pl.ds(cb, ch)], aidx)
        pltpu.sync_copy(eip_hbm.at[1, pl.ds(cb, ch)], bidx)

        def adj(g, _):
            bidx[pl.ds(g * 16, 16)] = bidx[pl.ds(g * 16, 16)] + 4096
            return 0
        lax.fori_loop(0, ch // 16, adj, 0, unroll=True)
        cp1 = pltpu.async_copy(fin_hbm.at[aidx], arows, sem1)
        cp2 = pltpu.async_copy(fin_hbm.at[bidx], brows, sem2)
        cp1.wait()
        cp2.wait()

        def dot1(e, _):
            acc = arows[e, pl.ds(0, 16)] * brows[e, pl.ds(0, 16)]
            for j in range(1, 16):
                acc = acc + arows[e, pl.ds(j * 16, 16)] * brows[e, pl.ds(j * 16, 16)]
            _sc_scalar_store(ybuf, e, jnp.sum(acc))
            return 0
        lax.fori_loop(0, ch, dot1, 0)
        pltpu.sync_copy(ybuf, y_hbm.at[pl.ds(cb, ch)])
        return 0
    lax.fori_loop(0, nch, chunk, 0)


def _pair_dot(fin_flat, eip):
    ep = eip.shape[1]
    ch = 128
    nch = ep // (_NW * ch)
    import functools as _ft
    f = _ft.partial(
        pl.kernel,
        out_type=jax.ShapeDtypeStruct((ep,), jnp.float32),
        mesh=plsc.VectorSubcoreMesh(**_SC_MESH),
        compiler_params=pltpu.CompilerParams(needs_layout_passes=False),
        scratch_types=[
            pltpu.VMEM((ch,), jnp.int32),
            pltpu.VMEM((ch,), jnp.int32),
            pltpu.VMEM((ch, 256), jnp.float32),
            pltpu.VMEM((ch, 256), jnp.float32),
            pltpu.VMEM((ch,), jnp.float32),
            pltpu.SemaphoreType.DMA,
            pltpu.SemaphoreType.DMA,
        ],
    )(_ft.partial(_pair_dot_body, ch=ch, nch=nch))
    return f(fin_flat, eip)


def _gather3_body(T_hbm, EI_hbm, G_hbm, sidx, didx, gidx, rows, sem, *, epw, ch):
    c = lax.axis_index("c")
    s = lax.axis_index("s")
    src_base = c * 12288
    dst_base = (1 - c) * 12288
    ebase = s * epw
    nch = epw // ch
    for role in range(3):
        def chunk(ci, _):
            cb = ebase + ci * ch
            pltpu.sync_copy(EI_hbm.at[c, 0, pl.ds(cb, ch)], sidx)
            pltpu.sync_copy(EI_hbm.at[c, 1, pl.ds(cb, ch)], didx)

            def bld(g, _):
                sl = pl.ds(g * 16, 16)
                if role == 0:
                    gidx[sl] = didx[sl] * 3 + dst_base
                else:
                    gidx[sl] = sidx[sl] * 3 + (src_base + role)
                return 0
            lax.fori_loop(0, ch // 16, bld, 0, unroll=True)
            pltpu.async_copy(T_hbm.at[gidx], rows, sem).wait()
            pltpu.sync_copy(rows, G_hbm.at[role, c, pl.ds(cb, ch)])
            return 0
        lax.fori_loop(0, nch, chunk, 0)


def _gather3(T, EI):
    epw = E // 16
    ch = 128
    f = functools.partial(
        pl.kernel,
        out_type=jax.ShapeDtypeStruct((3, 2, E, C), jnp.float32),
        mesh=plsc.VectorSubcoreMesh(**_SC_MESH),
        compiler_params=pltpu.CompilerParams(needs_layout_passes=False),
        scratch_types=[
            pltpu.VMEM((ch,), jnp.int32),
            pltpu.VMEM((ch,), jnp.int32),
            pltpu.VMEM((ch,), jnp.int32),
            pltpu.VMEM((ch, C), jnp.float32),
            pltpu.SemaphoreType.DMA,
        ],
    )(functools.partial(_gather3_body, epw=epw, ch=ch))
    return f(T, EI)


def _scatter_md(MSG, DEN, EI):
    epw = E // 16
    ch = 128
    w = 128

    def body(MSG_hbm, DEN_hbm, EI_hbm, U_hbm, D_hbm, didx, mchunk, dchunk,
             U_sh, D_sh, sem):
        c = lax.axis_index("c")
        s = lax.axis_index("s")
        ebase = s * epw
        nch = epw // ch
        zero16 = jnp.zeros((16,), jnp.float32)

        def zrow(r, _):
            for j in range(w // 16):
                mchunk[r, pl.ds(j * 16, 16)] = zero16
            return 0
        lax.fori_loop(0, ch, zrow, 0)
        rows_per_tile = 4096 // 16
        for k in range(rows_per_tile // ch):
            pltpu.sync_copy(mchunk, U_sh.at[pl.ds(s * rows_per_tile + k * ch, ch)])
            pltpu.sync_copy(mchunk, D_sh.at[pl.ds(s * rows_per_tile + k * ch, ch)])
        plsc.subcore_barrier()

        def chunk(ci, _):
            cb = ebase + ci * ch
            pltpu.sync_copy(MSG_hbm.at[c, pl.ds(cb, ch)], mchunk)
            pltpu.sync_copy(DEN_hbm.at[c, pl.ds(cb, ch)], dchunk)
            pltpu.sync_copy(EI_hbm.at[c, 1, pl.ds(cb, ch)], didx)
            pltpu.sync_copy(mchunk, U_sh.at[didx], add=True)
            pltpu.sync_copy(dchunk, D_sh.at[didx], add=True)
            return 0
        lax.fori_loop(0, nch, chunk, 0)

        plsc.subcore_barrier()
        for k in range(rows_per_tile // ch):
            off = s * rows_per_tile + k * ch
            pltpu.sync_copy(U_sh.at[pl.ds(off, ch)], U_hbm.at[c, pl.ds(off, ch)])
            pltpu.sync_copy(D_sh.at[pl.ds(off, ch)], D_hbm.at[c, pl.ds(off, ch)])

    f = pl.kernel(
        body,
        out_type=(jax.ShapeDtypeStruct((2, 4096, w), jnp.float32),
                  jax.ShapeDtypeStruct((2, 4096, w), jnp.float32)),
        mesh=plsc.VectorSubcoreMesh(**_SC_MESH),
        compiler_params=pltpu.CompilerParams(needs_layout_passes=False),
        scratch_types=[
            pltpu.VMEM((ch,), jnp.int32),
            pltpu.VMEM((ch, w), jnp.float32),
            pltpu.VMEM((ch, w), jnp.float32),
            pltpu.VMEM_SHARED((4096, w), jnp.float32),
            pltpu.VMEM_SHARED((4096, w), jnp.float32),
            pltpu.SemaphoreType.DMA,
        ],
    )
    return f(MSG, DEN, EI)


def _head_sum_mat():
    d = lax.broadcasted_iota(jnp.int32, (C, H), 0)
    h = lax.broadcasted_iota(jnp.int32, (C, H), 1)
    return (d // DH == h).astype(jnp.float32)


def _head_expand_mat():
    h = lax.broadcasted_iota(jnp.int32, (H, C), 0)
    d = lax.broadcasted_iota(jnp.int32, (H, C), 1)
    return (d // DH == h).astype(jnp.float32)


def _eye_pad_mat():
    h = lax.broadcasted_iota(jnp.int32, (H, C), 0)
    d = lax.broadcasted_iota(jnp.int32, (H, C), 1)
    return (d == h).astype(jnp.float32)


def _alpha_body(q_ref, k_ref, a_ref, m_ref):
    p = q_ref[0] * k_ref[0]
    alpha4 = jnp.dot(p, _head_sum_mat(), preferred_element_type=jnp.float32)
    a_ref[0] = alpha4
    m_ref[0, 0] = jnp.broadcast_to(jnp.max(alpha4, axis=0)[None, :], (8, H))


def _alpha_max(QE, KE, bm=4096):
    nb = E // bm
    return pl.pallas_call(
        _alpha_body,
        grid=(2, nb),
        in_specs=[
            pl.BlockSpec((1, bm, C), lambda t, i: (t, i, 0)),
            pl.BlockSpec((1, bm, C), lambda t, i: (t, i, 0)),
        ],
        out_specs=[
            pl.BlockSpec((1, bm, H), lambda t, i: (t, i, 0)),
            pl.BlockSpec((1, 1, 8, H), lambda t, i: (t, i, 0, 0)),
        ],
        out_shape=[
            jax.ShapeDtypeStruct((2, E, H), jnp.float32),
            jax.ShapeDtypeStruct((2, nb, 8, H), jnp.float32),
        ],
    )(QE, KE)


def _msgden_body(a_ref, v_ref, bm_ref, msg_ref, den_ref):
    m = jnp.max(bm_ref[0].reshape(-1, H), axis=0)
    ex = jnp.exp(a_ref[0] - m[None, :])
    msg_ref[0] = v_ref[0] * jnp.dot(ex, _head_expand_mat(),
                                    preferred_element_type=jnp.float32)
    den_ref[0] = jnp.dot(ex, _eye_pad_mat(), preferred_element_type=jnp.float32)


def _msg_den(alpha, VE, bmax, bm=4096):
    nb = E // bm
    return pl.pallas_call(
        _msgden_body,
        grid=(2, nb),
        in_specs=[
            pl.BlockSpec((1, bm, H), lambda t, i: (t, i, 0)),
            pl.BlockSpec((1, bm, C), lambda t, i: (t, i, 0)),
            pl.BlockSpec((1, nb, 8, H), lambda t, i: (t, 0, 0, 0)),
        ],
        out_specs=[
            pl.BlockSpec((1, bm, C), lambda t, i: (t, i, 0)),
            pl.BlockSpec((1, bm, C), lambda t, i: (t, i, 0)),
        ],
        out_shape=[
            jax.ShapeDtypeStruct((2, E, C), jnp.float32),
            jax.ShapeDtypeStruct((2, E, C), jnp.float32),
        ],
    )(alpha, VE, bmax)


def _fused_proj_weights(p, nt, et):
    Wq, bq = p[nt + "_q_W"], p[nt + "_q_b"]
    Wk, bk = p[nt + "_k_W"], p[nt + "_k_b"]
    Wv, bv = p[nt + "_v_W"], p[nt + "_v_b"]
    a_rel, m_rel, p_rel = p[et + "_a_rel"], p[et + "_m_rel"], p[et + "_p_rel"]
    scale = p_rel / math.sqrt(float(DH))
    Wkr = jnp.concatenate(
        [Wk[:, h*DH:(h+1)*DH] @ (a_rel[h] * scale[h]) for h in range(H)], axis=1)
    bkr = jnp.concatenate(
        [bk[h*DH:(h+1)*DH] @ (a_rel[h] * scale[h]) for h in range(H)])
    Wvr = jnp.concatenate(
        [Wv[:, h*DH:(h+1)*DH] @ m_rel[h] for h in range(H)], axis=1)
    bvr = jnp.concatenate([bv[h*DH:(h+1)*DH] @ m_rel[h] for h in range(H)])
    return jnp.concatenate([Wq, Wkr, Wvr], axis=1), jnp.concatenate([bq, bkr, bvr])


def _conv(x_dict, EI, p):
    W1, b1 = _fused_proj_weights(p, "n1", "n1__n2")
    W2, b2 = _fused_proj_weights(p, "n2", "n2__n1")
    P1 = _linear(x_dict["n1"], W1, b1)
    P2 = _linear(x_dict["n2"], W2, b2)
    T = jnp.concatenate([P1, P2], axis=0).reshape(6 * 4096, C)
    G = _gather3(T, EI)
    QE, KE, VE = G[0], G[1], G[2]
    alpha, bmax = _alpha_max(QE, KE)
    msg, den = _msg_den(alpha, VE, bmax)
    U, D = _scatter_md(msg, den, EI)
    out = {}
    for nt, et in (("n1", 1), ("n2", 0)):
        agg = U[et] / (jnp.repeat(D[et, :, :H], DH, axis=1) + 1e-16)
        o = jax.nn.gelu(agg)
        o = _linear(o, p[nt + "_a_W"], p[nt + "_a_b"])
        beta = jax.nn.sigmoid(p[nt + "_skip"])
        out[nt] = beta * o + (1.0 - beta) * x_dict[nt]
    return out


def kernel(x_n1, x_n2, edge_index_n1_n2, edge_index_n2_n1, edge_index,
           lin_n1_W, lin_n1_b, lin_n2_W, lin_n2_b, convs,
           lin_bias_n1_W, lin_bias_n1_b, lin_bias_n2_W, lin_bias_n2_b,
           convs_bias, x_bias_n1, x_bias_n2, att_bias):
    EI = jnp.stack([edge_index_n1_n2, edge_index_n2_n1])
    x = {"n1": _linear(x_n1, lin_n1_W, lin_n1_b, act="relu"),
         "n2": _linear(x_n2, lin_n2_W, lin_n2_b, act="relu")}
    allm = []
    for li in range(L):
        x = _conv(x, EI, convs[li])
        allm.append(x)
    xb = {"n1": _linear(x_bias_n1, lin_bias_n1_W, lin_bias_n1_b, act="relu"),
          "n2": _linear(x_bias_n2, lin_bias_n2_W, lin_bias_n2_b, act="relu")}
    allb = []
    for li in range(L):
        xb = _conv(xb, EI, convs_bias[li])
        allb.append(xb)
    ab = att_bias
    fin = {}
    for nt in NODE_TYPES:
        cm = jnp.concatenate([t[nt] for t in allm], axis=1)
        cb = jnp.concatenate([t[nt] for t in allb], axis=1)
        fin[nt] = ab[0] * cm + ab[1] * cb
    y_all = _final_matmul(fin["n1"], fin["n2"])
    fin_flat = jnp.concatenate([fin["n1"], fin["n2"]], axis=0)
    y = _pair_dot(fin_flat, edge_index).reshape(EP, 1)
    return y, y_all


# speedup vs baseline: 28.7323x; 1.0302x over previous
import functools
import math

import jax
import jax.numpy as jnp
from jax import lax
from jax.experimental import pallas as pl
from jax.experimental.pallas import tpu as pltpu
from jax.experimental.pallas import tpu_sc as plsc

N1 = 4096
N2 = 4096
D_FEAT = 128
C = 128
H = 4
DH = C // H
L = 2
HB = 64
E = 65536
EP = 16384
NODE_TYPES = ("n1", "n2")
EDGE_TYPES = (("n1", "n2"), ("n2", "n1"))
NUM_N = {"n1": N1, "n2": N2}


def _matmul_body(x_ref, w_ref, b_ref, o_ref, *, act):
    y = jnp.dot(x_ref[...], w_ref[...], preferred_element_type=jnp.float32)
    y = y + b_ref[...]
    if act == "relu":
        y = jnp.maximum(y, 0.0)
    o_ref[...] = y


def _linear(x, w, b, act=None, bm=512):
    n, d = x.shape
    dout = w.shape[1]
    return pl.pallas_call(
        functools.partial(_matmul_body, act=act),
        grid=(n // bm,),
        in_specs=[
            pl.BlockSpec((bm, d), lambda i: (i, 0)),
            pl.BlockSpec((d, dout), lambda i: (0, 0)),
            pl.BlockSpec((dout,), lambda i: (0,)),
        ],
        out_specs=pl.BlockSpec((bm, dout), lambda i: (i, 0)),
        out_shape=jax.ShapeDtypeStruct((n, dout), jnp.float32),
    )(x, w, b)


def _final_mm_body(a_ref, b_ref, o_ref):
    o_ref[...] = lax.dot_general(
        a_ref[...], b_ref[...], (((1,), (1,)), ((), ())),
        preferred_element_type=jnp.float32)


def _final_matmul(em, ed, bm=1024, bn=1024):
    n, k = em.shape
    m2 = ed.shape[0]
    return pl.pallas_call(
        _final_mm_body,
        grid=(n // bm, m2 // bn),
        in_specs=[
            pl.BlockSpec((bm, k), lambda i, j: (i, 0)),
            pl.BlockSpec((bn, k), lambda i, j: (j, 0)),
        ],
        out_specs=pl.BlockSpec((bm, bn), lambda i, j: (i, j)),
        out_shape=jax.ShapeDtypeStruct((n, m2), jnp.float32),
    )(em, ed)


_SC_MESH = dict(core_axis_name="c", subcore_axis_name="s")
_NW = 32


def _sc_scalar_store(ref, i, s):
    lanes = lax.iota(jnp.int32, 16)
    plsc.store_scatter(ref, [jnp.full((16,), i, jnp.int32)],
                       jnp.full((16,), s, jnp.float32), mask=lanes == 0)


def _pair_dot_body(fin_hbm, eip_hbm, y_hbm, aidx, bidx, arows, brows, ybuf,
                   sem1, sem2, *, ch, nch):
    wid = lax.axis_index("s") * 2 + lax.axis_index("c")
    base = wid * (ch * nch)

    def chunk(ci, _):
        cb = base + ci * ch
        pltpu.sync_copy(eip_hbm.at[0, pl.ds(cb, ch)], aidx)
        pltpu.sync_copy(eip_hbm.at[1, pl.ds(cb, ch)], bidx)

        def adj(g, _):
            bidx[pl.ds(g * 16, 16)] = bidx[pl.ds(g * 16, 16)] + 4096
            return 0
        lax.fori_loop(0, ch // 16, adj, 0, unroll=True)
        cp1 = pltpu.async_copy(fin_hbm.at[aidx], arows, sem1)
        cp2 = pltpu.async_copy(fin_hbm.at[bidx], brows, sem2)
        cp1.wait()
        cp2.wait()

        def dot1(e, _):
            acc = arows[e, pl.ds(0, 16)] * brows[e, pl.ds(0, 16)]
            for j in range(1, 16):
                acc = acc + arows[e, pl.ds(j * 16, 16)] * brows[e, pl.ds(j * 16, 16)]
            _sc_scalar_store(ybuf, e, jnp.sum(acc))
            return 0
        lax.fori_loop(0, ch, dot1, 0)
        pltpu.sync_copy(ybuf, y_hbm.at[pl.ds(cb, ch)])
        return 0
    lax.fori_loop(0, nch, chunk, 0)


def _pair_dot(fin_flat, eip):
    ep = eip.shape[1]
    ch = 128
    nch = ep // (_NW * ch)
    import functools as _ft
    f = _ft.partial(
        pl.kernel,
        out_type=jax.ShapeDtypeStruct((ep,), jnp.float32),
        mesh=plsc.VectorSubcoreMesh(**_SC_MESH),
        compiler_params=pltpu.CompilerParams(needs_layout_passes=False),
        scratch_types=[
            pltpu.VMEM((ch,), jnp.int32),
            pltpu.VMEM((ch,), jnp.int32),
            pltpu.VMEM((ch, 256), jnp.float32),
            pltpu.VMEM((ch, 256), jnp.float32),
            pltpu.VMEM((ch,), jnp.float32),
            pltpu.SemaphoreType.DMA,
            pltpu.SemaphoreType.DMA,
        ],
    )(_ft.partial(_pair_dot_body, ch=ch, nch=nch))
    return f(fin_flat, eip)


def _gather3_body(T_hbm, EI_hbm, G_hbm, sidx, didx, gidx, rows, rows2, sem,
                  wsem, *, epw, ch):
    c = lax.axis_index("c")
    s = lax.axis_index("s")
    src_base = c * 12288
    dst_base = (1 - c) * 12288
    ebase = s * epw
    nch = epw // ch
    for role in range(3):
        def chunk(ci, _):
            wbs = []
            for par, rbuf in ((0, rows), (1, rows2)):
                cb = ebase + (2 * ci + par) * ch
                pltpu.sync_copy(EI_hbm.at[c, 0, pl.ds(cb, ch)], sidx)
                pltpu.sync_copy(EI_hbm.at[c, 1, pl.ds(cb, ch)], didx)

                def bld(g, _):
                    sl = pl.ds(g * 16, 16)
                    if role == 0:
                        gidx[sl] = didx[sl] * 3 + dst_base
                    else:
                        gidx[sl] = sidx[sl] * 3 + (src_base + role)
                    return 0
                lax.fori_loop(0, ch // 16, bld, 0, unroll=True)
                pltpu.async_copy(T_hbm.at[gidx], rbuf, sem).wait()
                wbs.append(pltpu.async_copy(
                    rbuf, G_hbm.at[role, c, pl.ds(cb, ch)], wsem))
            wbs[0].wait()
            wbs[1].wait()
            return 0
        lax.fori_loop(0, nch // 2, chunk, 0)


def _gather3(T, EI):
    epw = E // 16
    ch = 128
    f = functools.partial(
        pl.kernel,
        out_type=jax.ShapeDtypeStruct((3, 2, E, C), jnp.float32),
        mesh=plsc.VectorSubcoreMesh(**_SC_MESH),
        compiler_params=pltpu.CompilerParams(needs_layout_passes=False),
        scratch_types=[
            pltpu.VMEM((ch,), jnp.int32),
            pltpu.VMEM((ch,), jnp.int32),
            pltpu.VMEM((ch,), jnp.int32),
            pltpu.VMEM((ch, C), jnp.float32),
            pltpu.VMEM((ch, C), jnp.float32),
            pltpu.SemaphoreType.DMA,
            pltpu.SemaphoreType.DMA,
        ],
    )(functools.partial(_gather3_body, epw=epw, ch=ch))
    return f(T, EI)


def _scatter_md(MSG, DEN, EI):
    epw = E // 16
    ch = 128
    w = 128

    def body(MSG_hbm, DEN_hbm, EI_hbm, U_hbm, D_hbm, didx, mchunk, dchunk,
             U_sh, D_sh, sem):
        c = lax.axis_index("c")
        s = lax.axis_index("s")
        ebase = s * epw
        nch = epw // ch
        zero16 = jnp.zeros((16,), jnp.float32)

        def zrow(r, _):
            for j in range(w // 16):
                mchunk[r, pl.ds(j * 16, 16)] = zero16
            return 0
        lax.fori_loop(0, ch, zrow, 0)
        rows_per_tile = 4096 // 16
        for k in range(rows_per_tile // ch):
            pltpu.sync_copy(mchunk, U_sh.at[pl.ds(s * rows_per_tile + k * ch, ch)])
            pltpu.sync_copy(mchunk, D_sh.at[pl.ds(s * rows_per_tile + k * ch, ch)])
        plsc.subcore_barrier()

        def chunk(ci, _):
            cb = ebase + ci * ch
            pltpu.sync_copy(MSG_hbm.at[c, pl.ds(cb, ch)], mchunk)
            pltpu.sync_copy(DEN_hbm.at[c, pl.ds(cb, ch)], dchunk)
            pltpu.sync_copy(EI_hbm.at[c, 1, pl.ds(cb, ch)], didx)
            pltpu.sync_copy(mchunk, U_sh.at[didx], add=True)
            pltpu.sync_copy(dchunk, D_sh.at[didx], add=True)
            return 0
        lax.fori_loop(0, nch, chunk, 0)

        plsc.subcore_barrier()
        for k in range(rows_per_tile // ch):
            off = s * rows_per_tile + k * ch
            pltpu.sync_copy(U_sh.at[pl.ds(off, ch)], U_hbm.at[c, pl.ds(off, ch)])
            pltpu.sync_copy(D_sh.at[pl.ds(off, ch)], D_hbm.at[c, pl.ds(off, ch)])

    f = pl.kernel(
        body,
        out_type=(jax.ShapeDtypeStruct((2, 4096, w), jnp.float32),
                  jax.ShapeDtypeStruct((2, 4096, w), jnp.float32)),
        mesh=plsc.VectorSubcoreMesh(**_SC_MESH),
        compiler_params=pltpu.CompilerParams(needs_layout_passes=False),
        scratch_types=[
            pltpu.VMEM((ch,), jnp.int32),
            pltpu.VMEM((ch, w), jnp.float32),
            pltpu.VMEM((ch, w), jnp.float32),
            pltpu.VMEM_SHARED((4096, w), jnp.float32),
            pltpu.VMEM_SHARED((4096, w), jnp.float32),
            pltpu.SemaphoreType.DMA,
        ],
    )
    return f(MSG, DEN, EI)


def _head_sum_mat():
    d = lax.broadcasted_iota(jnp.int32, (C, H), 0)
    h = lax.broadcasted_iota(jnp.int32, (C, H), 1)
    return (d // DH == h).astype(jnp.float32)


def _head_expand_mat():
    h = lax.broadcasted_iota(jnp.int32, (H, C), 0)
    d = lax.broadcasted_iota(jnp.int32, (H, C), 1)
    return (d // DH == h).astype(jnp.float32)


def _eye_pad_mat():
    h = lax.broadcasted_iota(jnp.int32, (H, C), 0)
    d = lax.broadcasted_iota(jnp.int32, (H, C), 1)
    return (d == h).astype(jnp.float32)


def _alpha_body(q_ref, k_ref, a_ref, m_ref):
    p = q_ref[0] * k_ref[0]
    alpha4 = jnp.dot(p, _head_sum_mat(), preferred_element_type=jnp.float32)
    a_ref[0] = alpha4
    m_ref[0, 0] = jnp.broadcast_to(jnp.max(alpha4, axis=0)[None, :], (8, H))


def _alpha_max(QE, KE, bm=4096):
    nb = E // bm
    return pl.pallas_call(
        _alpha_body,
        grid=(2, nb),
        in_specs=[
            pl.BlockSpec((1, bm, C), lambda t, i: (t, i, 0)),
            pl.BlockSpec((1, bm, C), lambda t, i: (t, i, 0)),
        ],
        out_specs=[
            pl.BlockSpec((1, bm, H), lambda t, i: (t, i, 0)),
            pl.BlockSpec((1, 1, 8, H), lambda t, i: (t, i, 0, 0)),
        ],
        out_shape=[
            jax.ShapeDtypeStruct((2, E, H), jnp.float32),
            jax.ShapeDtypeStruct((2, nb, 8, H), jnp.float32),
        ],
    )(QE, KE)


def _msgden_body(a_ref, v_ref, bm_ref, msg_ref, den_ref):
    m = jnp.max(bm_ref[0].reshape(-1, H), axis=0)
    ex = jnp.exp(a_ref[0] - m[None, :])
    msg_ref[0] = v_ref[0] * jnp.dot(ex, _head_expand_mat(),
                                    preferred_element_type=jnp.float32)
    den_ref[0] = jnp.dot(ex, _eye_pad_mat(), preferred_element_type=jnp.float32)


def _msg_den(alpha, VE, bmax, bm=4096):
    nb = E // bm
    return pl.pallas_call(
        _msgden_body,
        grid=(2, nb),
        in_specs=[
            pl.BlockSpec((1, bm, H), lambda t, i: (t, i, 0)),
            pl.BlockSpec((1, bm, C), lambda t, i: (t, i, 0)),
            pl.BlockSpec((1, nb, 8, H), lambda t, i: (t, 0, 0, 0)),
        ],
        out_specs=[
            pl.BlockSpec((1, bm, C), lambda t, i: (t, i, 0)),
            pl.BlockSpec((1, bm, C), lambda t, i: (t, i, 0)),
        ],
        out_shape=[
            jax.ShapeDtypeStruct((2, E, C), jnp.float32),
            jax.ShapeDtypeStruct((2, E, C), jnp.float32),
        ],
    )(alpha, VE, bmax)


def _fused_proj_weights(p, nt, et):
    Wq, bq = p[nt + "_q_W"], p[nt + "_q_b"]
    Wk, bk = p[nt + "_k_W"], p[nt + "_k_b"]
    Wv, bv = p[nt + "_v_W"], p[nt + "_v_b"]
    a_rel, m_rel, p_rel = p[et + "_a_rel"], p[et + "_m_rel"], p[et + "_p_rel"]
    scale = p_rel / math.sqrt(float(DH))
    Wkr = jnp.concatenate(
        [Wk[:, h*DH:(h+1)*DH] @ (a_rel[h] * scale[h]) for h in range(H)], axis=1)
    bkr = jnp.concatenate(
        [bk[h*DH:(h+1)*DH] @ (a_rel[h] * scale[h]) for h in range(H)])
    Wvr = jnp.concatenate(
        [Wv[:, h*DH:(h+1)*DH] @ m_rel[h] for h in range(H)], axis=1)
    bvr = jnp.concatenate([bv[h*DH:(h+1)*DH] @ m_rel[h] for h in range(H)])
    return jnp.concatenate([Wq, Wkr, Wvr], axis=1), jnp.concatenate([bq, bkr, bvr])


def _conv(x_dict, EI, p):
    W1, b1 = _fused_proj_weights(p, "n1", "n1__n2")
    W2, b2 = _fused_proj_weights(p, "n2", "n2__n1")
    P1 = _linear(x_dict["n1"], W1, b1)
    P2 = _linear(x_dict["n2"], W2, b2)
    T = jnp.concatenate([P1, P2], axis=0).reshape(6 * 4096, C)
    G = _gather3(T, EI)
    QE, KE, VE = G[0], G[1], G[2]
    alpha, bmax = _alpha_max(QE, KE)
    msg, den = _msg_den(alpha, VE, bmax)
    U, D = _scatter_md(msg, den, EI)
    out = {}
    for nt, et in (("n1", 1), ("n2", 0)):
        agg = U[et] / (jnp.repeat(D[et, :, :H], DH, axis=1) + 1e-16)
        o = jax.nn.gelu(agg)
        o = _linear(o, p[nt + "_a_W"], p[nt + "_a_b"])
        beta = jax.nn.sigmoid(p[nt + "_skip"])
        out[nt] = beta * o + (1.0 - beta) * x_dict[nt]
    return out


def kernel(x_n1, x_n2, edge_index_n1_n2, edge_index_n2_n1, edge_index,
           lin_n1_W, lin_n1_b, lin_n2_W, lin_n2_b, convs,
           lin_bias_n1_W, lin_bias_n1_b, lin_bias_n2_W, lin_bias_n2_b,
           convs_bias, x_bias_n1, x_bias_n2, att_bias):
    EI = jnp.stack([edge_index_n1_n2, edge_index_n2_n1])
    x = {"n1": _linear(x_n1, lin_n1_W, lin_n1_b, act="relu"),
         "n2": _linear(x_n2, lin_n2_W, lin_n2_b, act="relu")}
    allm = []
    for li in range(L):
        x = _conv(x, EI, convs[li])
        allm.append(x)
    xb = {"n1": _linear(x_bias_n1, lin_bias_n1_W, lin_bias_n1_b, act="relu"),
          "n2": _linear(x_bias_n2, lin_bias_n2_W, lin_bias_n2_b, act="relu")}
    allb = []
    for li in range(L):
        xb = _conv(xb, EI, convs_bias[li])
        allb.append(xb)
    ab = att_bias
    fin = {}
    for nt in NODE_TYPES:
        cm = jnp.concatenate([t[nt] for t in allm], axis=1)
        cb = jnp.concatenate([t[nt] for t in allb], axis=1)
        fin[nt] = ab[0] * cm + ab[1] * cb
    y_all = _final_matmul(fin["n1"], fin["n2"])
    fin_flat = jnp.concatenate([fin["n1"], fin["n2"]], axis=0)
    y = _pair_dot(fin_flat, edge_index).reshape(EP, 1)
    return y, y_all
